# baseline (speedup 1.0000x reference)
import jax
import jax.numpy as jnp
import numpy as np
from jax import lax
from jax.experimental import pallas as pl
from jax.experimental.pallas import tpu as pltpu
from jax.experimental.pallas import tpu_sc as plsc

_N = 10000
_E = 320000
_C = 128
_L = 2

_ROWS_PER_BLK = 1000
_TC_GRID = _N // _ROWS_PER_BLK

_K = 80
_NCHUNKS = _E // _K
_NCORES = 2
_NSUB = 16
_NTILES = _NCORES * _NSUB
_ECNT = _NCHUNKS // _NTILES

_PERM = np.arange(_C).reshape(_C // 32, 2, 16).transpose(0, 2, 1).reshape(_C)

_CW = _C + 16

_WCH = 80
_NWCH = _N // _WCH
_WFULL = _NWCH // _NSUB
_WREM = _NWCH - _WFULL * _NSUB


def _tf_body(h_ref, ws_ref, wr_ref, was_ref, wad_ref,
             m_ref, r_ref, as_ref, ad_ref):
    h = h_ref[...]
    m_ref[...] = jnp.dot(h, ws_ref[...],
                         preferred_element_type=jnp.float32).astype(jnp.bfloat16)
    r_ref[...] = jnp.dot(h, wr_ref[...], preferred_element_type=jnp.float32)
    as_ref[...] = jnp.dot(h, was_ref[...], preferred_element_type=jnp.float32)
    ad_ref[...] = jnp.dot(h, wad_ref[...], preferred_element_type=jnp.float32)


def _transform(h, ws, wr, was, wad):
    return pl.pallas_call(
        _tf_body,
        grid=(_TC_GRID,),
        in_specs=[
            pl.BlockSpec((_ROWS_PER_BLK, _C), lambda i: (i, 0)),
            pl.BlockSpec((_C, _C), lambda i: (0, 0)),
            pl.BlockSpec((_C, _C), lambda i: (0, 0)),
            pl.BlockSpec((_C, 16), lambda i: (0, 0)),
            pl.BlockSpec((_C, 16), lambda i: (0, 0)),
        ],
        out_specs=[
            pl.BlockSpec((_ROWS_PER_BLK, _C), lambda i: (i, 0)),
            pl.BlockSpec((_ROWS_PER_BLK, _C), lambda i: (i, 0)),
            pl.BlockSpec((_ROWS_PER_BLK, 16), lambda i: (i, 0)),
            pl.BlockSpec((_ROWS_PER_BLK, 16), lambda i: (i, 0)),
        ],
        out_shape=[
            jax.ShapeDtypeStruct((_N, _C), jnp.bfloat16),
            jax.ShapeDtypeStruct((_N, _C), jnp.float32),
            jax.ShapeDtypeStruct((_N, 16), jnp.float32),
            jax.ShapeDtypeStruct((_N, 16), jnp.float32),
        ],
    )(h, ws, wr, was, wad)


def _sc_body(m_hbm, asrc_hbm, adst_hbm, src_hbm, dst_hbm,
             num_hbm,
             agg_sh,
             idx3s, idx3d, asb2, adb2, rowsbf, srows,
             rsem, asem, dsem, ssem, issem, idsem):
    cid = lax.axis_index("c")
    sid = lax.axis_index("s")
    wid = cid * _NSUB + sid

    def _zrow(j, _):
        for k in range(9):
            srows[0, j, pl.ds(16 * k, 16)] = jnp.zeros((16,), jnp.float32)
            srows[1, j, pl.ds(16 * k, 16)] = jnp.zeros((16,), jnp.float32)
        return 0
    lax.fori_loop(0, _K, _zrow, 0)

    zbase = sid * _WFULL + jnp.minimum(sid, _WREM)
    zcnt = _WFULL + (sid < _WREM).astype(jnp.int32)

    def _zchunk(i, _):
        ck = zbase + i
        pltpu.sync_copy(srows.at[0, pl.ds(0, _WCH)],
                        agg_sh.at[pl.ds(ck * _WCH, _WCH)])
        return 0
    lax.fori_loop(0, zcnt, _zchunk, 0)

    plsc.subcore_barrier()

    ebase = wid * _ECNT

    def _issue_idx(slot, chunk):
        off = chunk * _K
        pltpu.async_copy(src_hbm.at[pl.ds(off, _K)], idx3s.at[slot],
                         issem.at[slot])
        pltpu.async_copy(dst_hbm.at[pl.ds(off, _K)], idx3d.at[slot],
                         idsem.at[slot])

    def _wait_idx(slot):
        pltpu.make_async_copy(src_hbm.at[pl.ds(0, _K)], idx3s.at[slot],
                              issem.at[slot]).wait()
        pltpu.make_async_copy(dst_hbm.at[pl.ds(0, _K)], idx3d.at[slot],
                              idsem.at[slot]).wait()

    def _issue_gather(par, slot):
        pltpu.async_copy(m_hbm.at[idx3s.at[slot]], rowsbf.at[par],
                         rsem.at[par])
        pltpu.async_copy(asrc_hbm.at[idx3s.at[slot]], asb2.at[par],
                         asem.at[par])
        pltpu.async_copy(adst_hbm.at[idx3d.at[slot]], adb2.at[par],
                         dsem.at[par])

    def _wait_gather(par):
        pltpu.make_async_copy(m_hbm.at[idx3s.at[0]], rowsbf.at[par],
                              rsem.at[par]).wait()
        pltpu.make_async_copy(asrc_hbm.at[idx3s.at[0]], asb2.at[par],
                              asem.at[par]).wait()
        pltpu.make_async_copy(adst_hbm.at[idx3d.at[0]], adb2.at[par],
                              dsem.at[par]).wait()

    def _issue_scatter(par, slot):
        pltpu.async_copy(srows.at[par], agg_sh.at[idx3d.at[slot]],
                         ssem.at[par], add=True)

    def _wait_scatter(par):
        pltpu.make_async_copy(srows.at[par], agg_sh.at[idx3d.at[0]],
                              ssem.at[par]).wait()

    pltpu.sync_copy(src_hbm.at[pl.ds(ebase * _K, _K)], idx3s.at[0])
    pltpu.sync_copy(dst_hbm.at[pl.ds(ebase * _K, _K)], idx3d.at[0])
    _issue_scatter(1, 0)
    _issue_gather(0, 0)
    _issue_idx(1, ebase + 1)

    def _chunk(i, _):
        par = lax.rem(i, 2)
        nxt = 1 - par
        t1 = lax.rem(i + 1, 3)
        t2 = lax.rem(i + 2, 3)
        _wait_gather(par)
        _wait_idx(t1)
        _wait_scatter(nxt)
        _issue_gather(nxt, t1)
        _issue_idx(t2, ebase + jnp.minimum(i + 2, _ECNT - 1))

        @plsc.parallel_loop(0, _K, 1, unroll=2)
        def _edge(j):
            e = asb2[par, j, pl.ds(0, 16)] + adb2[par, j, pl.ds(0, 16)]
            e = jnp.maximum(e, 0.2 * e)
            w = jnp.exp(e)
            srows[par, j, pl.ds(_C, 16)] = w
            for k in range(4):
                ab = rowsbf[par, j, pl.ds(32 * k, 32)]
                lo, hi = plsc.unpack(ab, format=plsc.PackFormat.INTERLEAVED)
                srows[par, j, pl.ds(32 * k, 16)] = lo * w
                srows[par, j, pl.ds(32 * k + 16, 16)] = hi * w

        _issue_scatter(par, lax.rem(i, 3))
        return 0

    lax.fori_loop(0, _ECNT, _chunk, 0)
    _wait_scatter(lax.rem(_ECNT - 1, 2))
    _wait_gather(lax.rem(_ECNT, 2))
    _wait_idx(lax.rem(_ECNT + 1, 3))
    plsc.subcore_barrier()

    def _wchunk(i, _):
        ck = zbase + i
        pltpu.sync_copy(agg_sh.at[pl.ds(ck * _WCH, _WCH)],
                        srows.at[0, pl.ds(0, _WCH)])
        pltpu.sync_copy(srows.at[0, pl.ds(0, _WCH)],
                        num_hbm.at[pl.ds(cid * _N + ck * _WCH, _WCH)])
        return 0
    lax.fori_loop(0, zcnt, _wchunk, 0)


def _sc_edge_pass(m, asrc_tab, adst_tab, src, dst):
    mesh = plsc.VectorSubcoreMesh(core_axis_name="c", subcore_axis_name="s")
    return pl.kernel(
        _sc_body,
        out_type=jax.ShapeDtypeStruct((_NCORES * _N, _CW), jnp.float32),
        mesh=mesh,
        compiler_params=pltpu.CompilerParams(use_tc_tiling_on_sc=False,
                                             needs_layout_passes=False),
        scratch_types=[
            pltpu.VMEM_SHARED((_N, _CW), jnp.float32),
            pltpu.VMEM((3, _K), jnp.int32),
            pltpu.VMEM((3, _K), jnp.int32),
            pltpu.VMEM((2, _K, 16), jnp.float32),
            pltpu.VMEM((2, _K, 16), jnp.float32),
            pltpu.VMEM((2, _K, _C), jnp.bfloat16),
            pltpu.VMEM((2, _K, _CW), jnp.float32),
            pltpu.SemaphoreType.DMA((2,)),
            pltpu.SemaphoreType.DMA((2,)),
            pltpu.SemaphoreType.DMA((2,)),
            pltpu.SemaphoreType.DMA((2,)),
            pltpu.SemaphoreType.DMA((3,)),
            pltpu.SemaphoreType.DMA((3,)),
        ],
    )(m, asrc_tab, adst_tab, src, dst)


def _fin_body(r_ref, a0_ref, a1_ref, g_ref, b_ref, o_ref):
    a0 = a0_ref[...]
    a1 = a1_ref[...]
    den = a0[:, _C:_C + 1] + a1[:, _C:_C + 1] + 1e-16
    out = r_ref[...] + (a0[:, :_C] + a1[:, :_C]) / den
    mu = jnp.mean(out, axis=-1, keepdims=True)
    var = jnp.mean((out - mu) ** 2, axis=-1, keepdims=True)
    y = (out - mu) * lax.rsqrt(var + 1e-5) * g_ref[...] + b_ref[...]
    o_ref[...] = jnp.maximum(y, 0.0)


def _finalize(r, comb, g, b):
    return pl.pallas_call(
        _fin_body,
        grid=(_TC_GRID,),
        in_specs=[
            pl.BlockSpec((_ROWS_PER_BLK, _C), lambda i: (i, 0)),
            pl.BlockSpec((_ROWS_PER_BLK, _CW), lambda i: (i, 0)),
            pl.BlockSpec((_ROWS_PER_BLK, _CW), lambda i: (i + _TC_GRID, 0)),
            pl.BlockSpec((1, _C), lambda i: (0, 0)),
            pl.BlockSpec((1, _C), lambda i: (0, 0)),
        ],
        out_specs=pl.BlockSpec((_ROWS_PER_BLK, _C), lambda i: (i, 0)),
        out_shape=jax.ShapeDtypeStruct((_N, _C), jnp.float32),
    )(r, comb, comb, g, b)


def _ft_body(r_ref, a0_ref, a1_ref, g_ref, b_ref,
             ws_ref, wr_ref, was_ref, wad_ref,
             m_ref, r2_ref, as_ref, ad_ref):
    a0 = a0_ref[...]
    a1 = a1_ref[...]
    den = a0[:, _C:_C + 1] + a1[:, _C:_C + 1] + 1e-16
    out = r_ref[...] + (a0[:, :_C] + a1[:, :_C]) / den
    mu = jnp.mean(out, axis=-1, keepdims=True)
    var = jnp.mean((out - mu) ** 2, axis=-1, keepdims=True)
    y = (out - mu) * lax.rsqrt(var + 1e-5) * g_ref[...] + b_ref[...]
    h = jnp.maximum(y, 0.0)
    m_ref[...] = jnp.dot(h, ws_ref[...],
                         preferred_element_type=jnp.float32).astype(jnp.bfloat16)
    r2_ref[...] = jnp.dot(h, wr_ref[...], preferred_element_type=jnp.float32)
    as_ref[...] = jnp.dot(h, was_ref[...], preferred_element_type=jnp.float32)
    ad_ref[...] = jnp.dot(h, wad_ref[...], preferred_element_type=jnp.float32)


def _fin_transform(r, comb, g, b, ws, wr, was, wad):
    return pl.pallas_call(
        _ft_body,
        grid=(_TC_GRID,),
        in_specs=[
            pl.BlockSpec((_ROWS_PER_BLK, _C), lambda i: (i, 0)),
            pl.BlockSpec((_ROWS_PER_BLK, _CW), lambda i: (i, 0)),
            pl.BlockSpec((_ROWS_PER_BLK, _CW), lambda i: (i + _TC_GRID, 0)),
            pl.BlockSpec((1, _C), lambda i: (0, 0)),
            pl.BlockSpec((1, _C), lambda i: (0, 0)),
            pl.BlockSpec((_C, _C), lambda i: (0, 0)),
            pl.BlockSpec((_C, _C), lambda i: (0, 0)),
            pl.BlockSpec((_C, 16), lambda i: (0, 0)),
            pl.BlockSpec((_C, 16), lambda i: (0, 0)),
        ],
        out_specs=[
            pl.BlockSpec((_ROWS_PER_BLK, _C), lambda i: (i, 0)),
            pl.BlockSpec((_ROWS_PER_BLK, _C), lambda i: (i, 0)),
            pl.BlockSpec((_ROWS_PER_BLK, 16), lambda i: (i, 0)),
            pl.BlockSpec((_ROWS_PER_BLK, 16), lambda i: (i, 0)),
        ],
        out_shape=[
            jax.ShapeDtypeStruct((_N, _C), jnp.bfloat16),
            jax.ShapeDtypeStruct((_N, _C), jnp.float32),
            jax.ShapeDtypeStruct((_N, 16), jnp.float32),
            jax.ShapeDtypeStruct((_N, 16), jnp.float32),
        ],
    )(r, comb, comb, g, b, ws, wr, was, wad)


def kernel(x, edge_index, W_src, W_root, a_src, a_dst, ln_scale, ln_bias):
    src = edge_index[0]
    dst = edge_index[1]
    was = [jnp.tile((W_src[i] @ a_src[i])[:, None], (1, 16)) for i in range(_L)]
    wad = [jnp.tile((W_root[i] @ a_dst[i])[:, None], (1, 16)) for i in range(_L)]
    m, r, asrc_tab, adst_tab = _transform(x, W_src[0][:, _PERM],
                                          W_root[0], was[0], wad[0])
    comb = _sc_edge_pass(m, asrc_tab, adst_tab, src, dst)
    m, r, asrc_tab, adst_tab = _fin_transform(
        r, comb, ln_scale[0].reshape(1, _C), ln_bias[0].reshape(1, _C),
        W_src[1][:, _PERM], W_root[1], was[1], wad[1])
    comb = _sc_edge_pass(m, asrc_tab, adst_tab, src, dst)
    return _finalize(r, comb,
                     ln_scale[1].reshape(1, _C), ln_bias[1].reshape(1, _C))


# speedup vs baseline: 1.0558x; 1.0558x over previous
import jax
import jax.numpy as jnp
import numpy as np
from jax import lax
from jax.experimental import pallas as pl
from jax.experimental.pallas import tpu as pltpu
from jax.experimental.pallas import tpu_sc as plsc

_N = 10000
_E = 320000
_C = 128
_L = 2

_ROWS_PER_BLK = 1000
_TC_GRID = _N // _ROWS_PER_BLK

_K = 80
_NCHUNKS = _E // _K
_NCORES = 2
_NSUB = 16
_NTILES = _NCORES * _NSUB
_ECNT = _NCHUNKS // _NTILES

_PERM = np.arange(_C).reshape(_C // 32, 2, 16).transpose(0, 2, 1).reshape(_C)

_CW = _C + 16

_WCH = 80
_NWCH = _N // _WCH
_WFULL = _NWCH // _NSUB
_WREM = _NWCH - _WFULL * _NSUB


def _tf_body(h_ref, ws_ref, wr_ref, was_ref, wad_ref,
             m_ref, r_ref, as_ref, ad_ref):
    h = h_ref[...]
    m_ref[...] = jnp.dot(h, ws_ref[...],
                         preferred_element_type=jnp.float32).astype(jnp.bfloat16)
    r_ref[...] = jnp.dot(h, wr_ref[...], preferred_element_type=jnp.float32)
    as_ref[...] = jnp.dot(h, was_ref[...], preferred_element_type=jnp.float32)
    ad_ref[...] = jnp.dot(h, wad_ref[...], preferred_element_type=jnp.float32)


def _transform(h, ws, wr, was, wad):
    return pl.pallas_call(
        _tf_body,
        grid=(_TC_GRID,),
        in_specs=[
            pl.BlockSpec((_ROWS_PER_BLK, _C), lambda i: (i, 0)),
            pl.BlockSpec((_C, _C), lambda i: (0, 0)),
            pl.BlockSpec((_C, _C), lambda i: (0, 0)),
            pl.BlockSpec((_C, 16), lambda i: (0, 0)),
            pl.BlockSpec((_C, 16), lambda i: (0, 0)),
        ],
        out_specs=[
            pl.BlockSpec((_ROWS_PER_BLK, _C), lambda i: (i, 0)),
            pl.BlockSpec((_ROWS_PER_BLK, _C), lambda i: (i, 0)),
            pl.BlockSpec((_ROWS_PER_BLK, 16), lambda i: (i, 0)),
            pl.BlockSpec((_ROWS_PER_BLK, 16), lambda i: (i, 0)),
        ],
        out_shape=[
            jax.ShapeDtypeStruct((_N, _C), jnp.bfloat16),
            jax.ShapeDtypeStruct((_N, _C), jnp.float32),
            jax.ShapeDtypeStruct((_N, 16), jnp.float32),
            jax.ShapeDtypeStruct((_N, 16), jnp.float32),
        ],
    )(h, ws, wr, was, wad)


def _sc_body(m_hbm, asrc_hbm, adst_hbm, src_hbm, dst_hbm,
             num_hbm, den_hbm,
             agg_sh, den_sh,
             idx3s, idx3d, asb2, adb2, ew2, rowsbf, srows,
             rsem, asem, dsem, ssem, issem, idsem):
    cid = lax.axis_index("c")
    sid = lax.axis_index("s")
    wid = cid * _NSUB + sid

    def _zrow(j, _):
        for k in range(8):
            srows[0, j, pl.ds(16 * k, 16)] = jnp.zeros((16,), jnp.float32)
            srows[1, j, pl.ds(16 * k, 16)] = jnp.zeros((16,), jnp.float32)
        ew2[0, j, pl.ds(0, 16)] = jnp.zeros((16,), jnp.float32)
        ew2[1, j, pl.ds(0, 16)] = jnp.zeros((16,), jnp.float32)
        return 0
    lax.fori_loop(0, _K, _zrow, 0)

    zbase = sid * _WFULL + jnp.minimum(sid, _WREM)
    zcnt = _WFULL + (sid < _WREM).astype(jnp.int32)

    def _zchunk(i, _):
        ck = zbase + i
        pltpu.sync_copy(srows.at[0, pl.ds(0, _WCH)],
                        agg_sh.at[pl.ds(ck * _WCH, _WCH)])
        pltpu.sync_copy(ew2.at[0, pl.ds(0, _WCH)],
                        den_sh.at[pl.ds(ck * _WCH, _WCH)])
        return 0
    lax.fori_loop(0, zcnt, _zchunk, 0)

    plsc.subcore_barrier()

    ebase = wid * _ECNT

    def _issue_idx(slot, chunk):
        off = chunk * _K
        pltpu.async_copy(src_hbm.at[pl.ds(off, _K)], idx3s.at[slot],
                         issem.at[slot])
        pltpu.async_copy(dst_hbm.at[pl.ds(off, _K)], idx3d.at[slot],
                         idsem.at[slot])

    def _wait_idx(slot):
        pltpu.make_async_copy(src_hbm.at[pl.ds(0, _K)], idx3s.at[slot],
                              issem.at[slot]).wait()
        pltpu.make_async_copy(dst_hbm.at[pl.ds(0, _K)], idx3d.at[slot],
                              idsem.at[slot]).wait()

    def _issue_gather(par, slot):
        pltpu.async_copy(m_hbm.at[idx3s.at[slot]], rowsbf.at[par],
                         rsem.at[par])
        pltpu.async_copy(asrc_hbm.at[idx3s.at[slot]], asb2.at[par],
                         asem.at[par])
        pltpu.async_copy(adst_hbm.at[idx3d.at[slot]], adb2.at[par],
                         dsem.at[par])

    def _wait_gather(par):
        pltpu.make_async_copy(m_hbm.at[idx3s.at[0]], rowsbf.at[par],
                              rsem.at[par]).wait()
        pltpu.make_async_copy(asrc_hbm.at[idx3s.at[0]], asb2.at[par],
                              asem.at[par]).wait()
        pltpu.make_async_copy(adst_hbm.at[idx3d.at[0]], adb2.at[par],
                              dsem.at[par]).wait()

    def _issue_scatter(par, slot):
        pltpu.async_copy(srows.at[par], agg_sh.at[idx3d.at[slot]],
                         ssem.at[par], add=True)
        pltpu.async_copy(ew2.at[par], den_sh.at[idx3d.at[slot]],
                         ssem.at[par], add=True)

    def _wait_scatter(par):
        pltpu.make_async_copy(srows.at[par], agg_sh.at[idx3d.at[0]],
                              ssem.at[par]).wait()
        pltpu.make_async_copy(ew2.at[par], den_sh.at[idx3d.at[0]],
                              ssem.at[par]).wait()

    pltpu.sync_copy(src_hbm.at[pl.ds(ebase * _K, _K)], idx3s.at[0])
    pltpu.sync_copy(dst_hbm.at[pl.ds(ebase * _K, _K)], idx3d.at[0])
    _issue_scatter(1, 0)
    _issue_gather(0, 0)
    _issue_idx(1, ebase + 1)

    def _chunk(i, _):
        par = lax.rem(i, 2)
        nxt = 1 - par
        t1 = lax.rem(i + 1, 3)
        t2 = lax.rem(i + 2, 3)
        _wait_gather(par)
        _wait_idx(t1)
        _wait_scatter(nxt)
        _issue_gather(nxt, t1)
        _issue_idx(t2, ebase + jnp.minimum(i + 2, _ECNT - 1))

        @plsc.parallel_loop(0, _K, 1, unroll=4)
        def _edge(j):
            e = asb2[par, j, pl.ds(0, 16)] + adb2[par, j, pl.ds(0, 16)]
            e = jnp.maximum(e, 0.2 * e)
            w = jnp.exp(e)
            ew2[par, j, pl.ds(0, 16)] = w
            for k in range(4):
                ab = rowsbf[par, j, pl.ds(32 * k, 32)]
                lo, hi = plsc.unpack(ab, format=plsc.PackFormat.INTERLEAVED)
                srows[par, j, pl.ds(32 * k, 16)] = lo * w
                srows[par, j, pl.ds(32 * k + 16, 16)] = hi * w

        _issue_scatter(par, lax.rem(i, 3))
        return 0

    lax.fori_loop(0, _ECNT, _chunk, 0)
    _wait_scatter(lax.rem(_ECNT - 1, 2))
    _wait_gather(lax.rem(_ECNT, 2))
    _wait_idx(lax.rem(_ECNT + 1, 3))
    plsc.subcore_barrier()

    def _wchunk(i, _):
        ck = zbase + i
        pltpu.sync_copy(agg_sh.at[pl.ds(ck * _WCH, _WCH)],
                        srows.at[0, pl.ds(0, _WCH)])
        pltpu.sync_copy(srows.at[0, pl.ds(0, _WCH)],
                        num_hbm.at[pl.ds(cid * _N + ck * _WCH, _WCH)])
        pltpu.sync_copy(den_sh.at[pl.ds(ck * _WCH, _WCH)],
                        ew2.at[0, pl.ds(0, _WCH)])
        pltpu.sync_copy(ew2.at[0, pl.ds(0, _WCH)],
                        den_hbm.at[pl.ds(cid * _N + ck * _WCH, _WCH)])
        return 0
    lax.fori_loop(0, zcnt, _wchunk, 0)


def _sc_edge_pass(m, asrc_tab, adst_tab, src, dst):
    mesh = plsc.VectorSubcoreMesh(core_axis_name="c", subcore_axis_name="s")
    return pl.kernel(
        _sc_body,
        out_type=[
            jax.ShapeDtypeStruct((_NCORES * _N, _C), jnp.float32),
            jax.ShapeDtypeStruct((_NCORES * _N, 16), jnp.float32),
        ],
        mesh=mesh,
        compiler_params=pltpu.CompilerParams(use_tc_tiling_on_sc=False,
                                             needs_layout_passes=False),
        scratch_types=[
            pltpu.VMEM_SHARED((_N, _C), jnp.float32),
            pltpu.VMEM_SHARED((_N, 16), jnp.float32),
            pltpu.VMEM((3, _K), jnp.int32),
            pltpu.VMEM((3, _K), jnp.int32),
            pltpu.VMEM((2, _K, 16), jnp.float32),
            pltpu.VMEM((2, _K, 16), jnp.float32),
            pltpu.VMEM((2, _K, 16), jnp.float32),
            pltpu.VMEM((2, _K, _C), jnp.bfloat16),
            pltpu.VMEM((2, _K, _C), jnp.float32),
            pltpu.SemaphoreType.DMA((2,)),
            pltpu.SemaphoreType.DMA((2,)),
            pltpu.SemaphoreType.DMA((2,)),
            pltpu.SemaphoreType.DMA((2,)),
            pltpu.SemaphoreType.DMA((3,)),
            pltpu.SemaphoreType.DMA((3,)),
        ],
    )(m, asrc_tab, adst_tab, src, dst)


def _fin_body(r_ref, a0_ref, a1_ref, d0_ref, d1_ref, g_ref, b_ref, o_ref):
    den = d0_ref[...][:, 0:1] + d1_ref[...][:, 0:1] + 1e-16
    out = r_ref[...] + (a0_ref[...] + a1_ref[...]) / den
    mu = jnp.mean(out, axis=-1, keepdims=True)
    var = jnp.mean((out - mu) ** 2, axis=-1, keepdims=True)
    y = (out - mu) * lax.rsqrt(var + 1e-5) * g_ref[...] + b_ref[...]
    o_ref[...] = jnp.maximum(y, 0.0)


def _finalize(r, num2, den2, g, b):
    return pl.pallas_call(
        _fin_body,
        grid=(_TC_GRID,),
        in_specs=[
            pl.BlockSpec((_ROWS_PER_BLK, _C), lambda i: (i, 0)),
            pl.BlockSpec((_ROWS_PER_BLK, _C), lambda i: (i, 0)),
            pl.BlockSpec((_ROWS_PER_BLK, _C), lambda i: (i + _TC_GRID, 0)),
            pl.BlockSpec((_ROWS_PER_BLK, 16), lambda i: (i, 0)),
            pl.BlockSpec((_ROWS_PER_BLK, 16), lambda i: (i + _TC_GRID, 0)),
            pl.BlockSpec((1, _C), lambda i: (0, 0)),
            pl.BlockSpec((1, _C), lambda i: (0, 0)),
        ],
        out_specs=pl.BlockSpec((_ROWS_PER_BLK, _C), lambda i: (i, 0)),
        out_shape=jax.ShapeDtypeStruct((_N, _C), jnp.float32),
    )(r, num2, num2, den2, den2, g, b)


def _ft_body(r_ref, a0_ref, a1_ref, d0_ref, d1_ref, g_ref, b_ref,
             ws_ref, wr_ref, was_ref, wad_ref,
             m_ref, r2_ref, as_ref, ad_ref):
    den = d0_ref[...][:, 0:1] + d1_ref[...][:, 0:1] + 1e-16
    out = r_ref[...] + (a0_ref[...] + a1_ref[...]) / den
    mu = jnp.mean(out, axis=-1, keepdims=True)
    var = jnp.mean((out - mu) ** 2, axis=-1, keepdims=True)
    y = (out - mu) * lax.rsqrt(var + 1e-5) * g_ref[...] + b_ref[...]
    h = jnp.maximum(y, 0.0)
    m_ref[...] = jnp.dot(h, ws_ref[...],
                         preferred_element_type=jnp.float32).astype(jnp.bfloat16)
    r2_ref[...] = jnp.dot(h, wr_ref[...], preferred_element_type=jnp.float32)
    as_ref[...] = jnp.dot(h, was_ref[...], preferred_element_type=jnp.float32)
    ad_ref[...] = jnp.dot(h, wad_ref[...], preferred_element_type=jnp.float32)


def _fin_transform(r, num2, den2, g, b, ws, wr, was, wad):
    return pl.pallas_call(
        _ft_body,
        grid=(_TC_GRID,),
        in_specs=[
            pl.BlockSpec((_ROWS_PER_BLK, _C), lambda i: (i, 0)),
            pl.BlockSpec((_ROWS_PER_BLK, _C), lambda i: (i, 0)),
            pl.BlockSpec((_ROWS_PER_BLK, _C), lambda i: (i + _TC_GRID, 0)),
            pl.BlockSpec((_ROWS_PER_BLK, 16), lambda i: (i, 0)),
            pl.BlockSpec((_ROWS_PER_BLK, 16), lambda i: (i + _TC_GRID, 0)),
            pl.BlockSpec((1, _C), lambda i: (0, 0)),
            pl.BlockSpec((1, _C), lambda i: (0, 0)),
            pl.BlockSpec((_C, _C), lambda i: (0, 0)),
            pl.BlockSpec((_C, _C), lambda i: (0, 0)),
            pl.BlockSpec((_C, 16), lambda i: (0, 0)),
            pl.BlockSpec((_C, 16), lambda i: (0, 0)),
        ],
        out_specs=[
            pl.BlockSpec((_ROWS_PER_BLK, _C), lambda i: (i, 0)),
            pl.BlockSpec((_ROWS_PER_BLK, _C), lambda i: (i, 0)),
            pl.BlockSpec((_ROWS_PER_BLK, 16), lambda i: (i, 0)),
            pl.BlockSpec((_ROWS_PER_BLK, 16), lambda i: (i, 0)),
        ],
        out_shape=[
            jax.ShapeDtypeStruct((_N, _C), jnp.bfloat16),
            jax.ShapeDtypeStruct((_N, _C), jnp.float32),
            jax.ShapeDtypeStruct((_N, 16), jnp.float32),
            jax.ShapeDtypeStruct((_N, 16), jnp.float32),
        ],
    )(r, num2, num2, den2, den2, g, b, ws, wr, was, wad)


def kernel(x, edge_index, W_src, W_root, a_src, a_dst, ln_scale, ln_bias):
    src = edge_index[0]
    dst = edge_index[1]
    was = [jnp.tile((W_src[i] @ a_src[i])[:, None], (1, 16)) for i in range(_L)]
    wad = [jnp.tile((W_root[i] @ a_dst[i])[:, None], (1, 16)) for i in range(_L)]
    m, r, asrc_tab, adst_tab = _transform(x, W_src[0][:, _PERM],
                                          W_root[0], was[0], wad[0])
    num2, den2 = _sc_edge_pass(m, asrc_tab, adst_tab, src, dst)
    m, r, asrc_tab, adst_tab = _fin_transform(
        r, num2, den2, ln_scale[0].reshape(1, _C), ln_bias[0].reshape(1, _C),
        W_src[1][:, _PERM], W_root[1], was[1], wad[1])
    num2, den2 = _sc_edge_pass(m, asrc_tab, adst_tab, src, dst)
    return _finalize(r, num2, den2,
                     ln_scale[1].reshape(1, _C), ln_bias[1].reshape(1, _C))
